# Initial kernel scaffold; baseline (speedup 1.0000x reference)
import jax
import jax.numpy as jnp
from jax.experimental import pallas as pl


def kernel(x, cache, wm, Wq_ltm, Wo_ltm, Wg_ltm, Wq_wm, Wo_wm, Wg_wm, conv_w1, conv_b1, conv_w2, conv_b2, post_g, post_b, Wwc_wm, Wws_wm, Wgw_wm, Wwc_ltm, Wws_ltm, Wgw_ltm):
    raise NotImplementedError("write your pallas kernel here")


# speedup vs baseline: 1.8467x; 1.8467x over previous
import jax
import jax.numpy as jnp
from jax.experimental import pallas as pl
from jax.experimental.pallas import tpu as pltpu

DM = 1024
DC = 256
NS = 768
NW = 8
KS = 5
TS = 512


def _dot(a, b):
    return jax.lax.dot_general(a.astype(jnp.bfloat16), b.astype(jnp.bfloat16),
                               (((a.ndim - 1,), (0,)), ((), ())),
                               preferred_element_type=jnp.float32)


def _dotg(a, b, dims):
    return jax.lax.dot_general(a.astype(jnp.bfloat16), b.astype(jnp.bfloat16),
                               dims, preferred_element_type=jnp.float32)


def _lnorm(x, g=None, b=None, eps=1e-5):
    m = jnp.mean(x, axis=-1, keepdims=True)
    v = jnp.mean((x - m) ** 2, axis=-1, keepdims=True)
    y = (x - m) * jax.lax.rsqrt(v + eps)
    if g is not None:
        y = y * g + b
    return y


def _softmax(s):
    s = s - jnp.max(s, axis=-1, keepdims=True)
    e = jnp.exp(s)
    return e / jnp.sum(e, axis=-1, keepdims=True)


def _proj_body(x_ref, w_ref, o_ref):
    o_ref[...] = _dot(x_ref[...], w_ref[...])


def _proj_tanh_body(x_ref, w_ref, o_ref):
    o_ref[...] = jnp.tanh(_dot(x_ref[...], w_ref[...]))


def _attn_body(q_ref, c_ref, a_ref):
    s = _dotg(q_ref[0], c_ref[0], (((1,), (1,)), ((), ()))) * (1.0 / 16.0)
    a_ref[0] = _softmax(s)


def _xltm_body(read_ref, x_ref, g_ref, wo_ref, o_ref):
    o_ref[0] = x_ref[0] + jax.nn.sigmoid(g_ref[0]) * _dot(read_ref[0], wo_ref[...])


def _wm_read_body(qw_ref, kv_ref, val_ref, xl_ref, g_ref, wo_ref, o_ref):
    kv = kv_ref[0]
    sw = _dotg(qw_ref[0], kv, (((1,), (1,)), ((), ()))) * (1.0 / 16.0)
    aw = _softmax(sw) * jnp.clip(val_ref[0], 0.0, 1.0)
    readw = _dot(aw, kv)
    o_ref[0] = xl_ref[0] + jax.nn.sigmoid(g_ref[0]) * _dot(readw, wo_ref[...])


def _conv_body(dil, xc_ref, xp_ref, w_ref, b_ref, o_ref, ext_ref):
    st = pl.program_id(1)
    xc = xc_ref[0]
    u = _lnorm(xc)
    halo = _lnorm(xp_ref[0][TS - 8:])
    ext_ref[0:8, :] = jnp.where(st == 0, 0.0, halo)
    ext_ref[8:, :] = u
    acc = _dot(ext_ref[pl.ds(8 - 4 * dil, TS), :], w_ref[0])
    for t in range(1, KS):
        off = 8 - 4 * dil + t * dil
        acc = acc + _dot(ext_ref[pl.ds(off, TS), :], w_ref[t])
    o_ref[0] = xc + jax.nn.gelu(acc + b_ref[0])


def _postln_body(h_ref, pg_ref, pb_ref, o_ref):
    o_ref[0] = _lnorm(h_ref[0], pg_ref[0], pb_ref[0])


def _accum_body(c_ref, l_ref, g_ref, cl_ref, sl_ref, gl_ref, cache_ref,
                ucache_ref, wmacc_ref, acc_ref):
    st = pl.program_id(1)
    nst = pl.num_programs(1)
    ones = jnp.ones((TS, 128), jnp.float32)
    logits = l_ref[0]
    g = jax.nn.sigmoid(g_ref[0])
    iota = jax.lax.broadcasted_iota(jnp.int32, (TS, NW), 1)
    m = jnp.max(logits, axis=-1, keepdims=True)
    first = jnp.min(jnp.where(logits == m, iota, NW), axis=-1, keepdims=True)
    w_wm = jnp.where(iota == first, g, 0.0)
    c_ext = jnp.concatenate([c_ref[0], ones], axis=1)
    contrib = _dotg(w_wm, c_ext, (((0,), (0,)), ((), ())))

    @pl.when(st == 0)
    def _():
        wmacc_ref[0] = contrib

    @pl.when(st != 0)
    def _():
        wmacc_ref[0] += contrib

    al = _softmax(sl_ref[0] * (1.0 / 16.0))
    wt = al * jax.nn.sigmoid(gl_ref[0])
    cl_ext = jnp.concatenate([cl_ref[0], ones], axis=1)
    lcontrib = _dotg(wt, cl_ext, (((0,), (0,)), ((), ())))

    @pl.when(st == 0)
    def _():
        acc_ref[...] = lcontrib

    @pl.when(st != 0)
    def _():
        acc_ref[...] += lcontrib

    @pl.when(st == nst - 1)
    def _():
        acc = acc_ref[...]
        numl = acc[:, :DC]
        denl = acc[:, DC:DC + 1]
        bl = denl / (denl + 1.0)
        ucache_ref[0] = (1.0 - bl) * cache_ref[0] + bl * (numl / (denl + 1e-6))


def kernel(x, cache, wm, Wq_ltm, Wo_ltm, Wg_ltm, Wq_wm, Wo_wm, Wg_wm,
           conv_w1, conv_b1, conv_w2, conv_b2, post_g, post_b,
           Wwc_wm, Wws_wm, Wgw_wm, Wwc_ltm, Wws_ltm, Wgw_ltm):
    B, S, _ = x.shape
    M = B * S
    ST = S // TS
    kv = wm[..., :DC]
    val = wm[..., DC:].reshape(B, 1, NW)
    b1 = conv_b1.reshape(1, DM)
    b2 = conv_b2.reshape(1, DM)
    pg = post_g.reshape(1, DM)
    pb = post_b.reshape(1, DM)
    f32 = jnp.float32

    def fullspec(shape):
        nd = len(shape)
        return pl.BlockSpec(shape, lambda: (0,) * nd)

    def wspec2(shape):
        nd = len(shape)
        return pl.BlockSpec(shape, lambda b, s: (0,) * nd)

    def proj(x2, w, body=_proj_body):
        n = w.shape[1]
        return pl.pallas_call(
            body,
            in_specs=[fullspec((M, DM)), fullspec((DM, n))],
            out_specs=fullspec((M, n)),
            out_shape=jax.ShapeDtypeStruct((M, n), f32),
        )(x2, w)

    x2 = x.reshape(M, DM)
    q2 = proj(x2, Wq_ltm)
    gl2 = proj(x2, Wg_ltm)

    bspec = lambda n: pl.BlockSpec((1, S, n), lambda b: (b, 0, 0))
    attn = pl.pallas_call(
        _attn_body, grid=(B,),
        in_specs=[bspec(DC), pl.BlockSpec((1, NS, DC), lambda b: (b, 0, 0))],
        out_specs=bspec(NS),
        out_shape=jax.ShapeDtypeStruct((B, S, NS), f32),
    )(q2.reshape(B, S, DC), cache)
    read = jnp.einsum('bsk,bkd->bsd', attn, cache)
    x_ltm = pl.pallas_call(
        _xltm_body, grid=(B,),
        in_specs=[bspec(DC), bspec(DM), bspec(1),
                  pl.BlockSpec((DC, DM), lambda b: (0, 0))],
        out_specs=bspec(DM),
        out_shape=jax.ShapeDtypeStruct((B, S, DM), f32),
    )(read, x, gl2.reshape(B, S, 1), Wo_ltm)

    xl2 = x_ltm.reshape(M, DM)
    qw2 = proj(xl2, Wq_wm)
    gw2 = proj(xl2, Wg_wm)

    x_enh = pl.pallas_call(
        _wm_read_body, grid=(B,),
        in_specs=[bspec(DC), pl.BlockSpec((1, NW, DC), lambda b: (b, 0, 0)),
                  pl.BlockSpec((1, 1, NW), lambda b: (b, 0, 0)),
                  bspec(DM), bspec(1), pl.BlockSpec((DC, DM), lambda b: (0, 0))],
        out_specs=bspec(DM),
        out_shape=jax.ShapeDtypeStruct((B, S, DM), f32),
    )(qw2.reshape(B, S, DC), kv, val, x_ltm, gw2.reshape(B, S, 1), Wo_wm)

    tile_spec = pl.BlockSpec((1, TS, DM), lambda b, s: (b, s, 0))
    prev_spec = pl.BlockSpec((1, TS, DM), lambda b, s: (b, jnp.maximum(s - 1, 0), 0))

    def conv(h, w, b, dil):
        return pl.pallas_call(
            lambda *refs: _conv_body(dil, *refs),
            grid=(B, ST),
            in_specs=[tile_spec, prev_spec, wspec2((KS, DM, DM)), wspec2((1, DM))],
            out_specs=tile_spec,
            out_shape=jax.ShapeDtypeStruct((B, S, DM), f32),
            scratch_shapes=[pltpu.VMEM((TS + 8, DM), f32)],
        )(h, h, w, b)

    h1 = conv(x_enh, conv_w1, b1, 1)
    h2 = conv(h1, conv_w2, b2, 2)

    output = pl.pallas_call(
        _postln_body, grid=(B, ST),
        in_specs=[tile_spec, wspec2((1, DM)), wspec2((1, DM))],
        out_specs=tile_spec,
        out_shape=jax.ShapeDtypeStruct((B, S, DM), f32),
    )(h2, pg, pb)

    out2 = output.reshape(M, DM)
    c2 = proj(out2, Wwc_wm, _proj_tanh_body)
    logits2 = proj(out2, Wws_wm)
    g2 = proj(out2, Wgw_wm)
    cl2 = proj(out2, Wwc_ltm, _proj_tanh_body)
    glw2 = proj(out2, Wgw_ltm)

    NC = NS // 6
    sl2 = pl.pallas_call(
        _proj_body, grid=(6,),
        in_specs=[pl.BlockSpec((M, DM), lambda i: (0, 0)),
                  pl.BlockSpec((DM, NC), lambda i: (0, i))],
        out_specs=pl.BlockSpec((M, NC), lambda i: (0, i)),
        out_shape=jax.ShapeDtypeStruct((M, NS), f32),
    )(out2, Wws_ltm)

    tspec = lambda n: pl.BlockSpec((1, TS, n), lambda b, s: (b, s, 0))
    cache_spec = pl.BlockSpec((1, NS, DC), lambda b, s: (b, 0, 0))
    ucache, wmacc = pl.pallas_call(
        _accum_body, grid=(B, ST),
        in_specs=[tspec(DC), tspec(NW), tspec(1), tspec(DC), tspec(NS),
                  tspec(1), cache_spec],
        out_specs=[cache_spec,
                   pl.BlockSpec((1, NW, DC + 128), lambda b, s: (b, 0, 0))],
        out_shape=[jax.ShapeDtypeStruct((B, NS, DC), f32),
                   jax.ShapeDtypeStruct((B, NW, DC + 128), f32)],
        scratch_shapes=[pltpu.VMEM((NS, DC + 128), f32)],
    )(c2.reshape(B, S, DC), logits2.reshape(B, S, NW), g2.reshape(B, S, 1),
      cl2.reshape(B, S, DC), sl2.reshape(B, S, NS), glw2.reshape(B, S, 1),
      cache)

    numer = wmacc[..., :DC]
    denom = wmacc[..., DC:DC + 1]
    alpha = denom / (denom + 1.0)
    new_kv = (1.0 - alpha) * kv + alpha * (numer / (denom + 1e-6))
    new_val = jnp.clip(wm[..., DC:] + alpha, 0.0, 1.0)
    updated_wm = jnp.concatenate([new_kv, new_val], axis=-1)
    return (output, ucache, updated_wm)
